# baseline (speedup 1.0000x reference)
import functools

import jax
import jax.numpy as jnp
from jax import lax
from jax.experimental import pallas as pl
from jax.experimental.pallas import tpu as pltpu
from jax.experimental.pallas import tpu_sc as plsc

_NC = 2
_NS = 16
_NW = _NC * _NS
_LANES = 16


def _table_mlp(features, w, b):
    n, d = features.shape
    rb = 5000
    assert n % 2 == 0 and (n // 2) % rb == 0

    def pack_half(t, d):
        h = d // 2
        a16 = t[:, :h].astype(jnp.bfloat16).astype(jnp.float32)
        b16 = t[:, h:].astype(jnp.bfloat16).astype(jnp.float32)
        au = jax.lax.bitcast_convert_type(a16, jnp.uint32) >> 16
        bu = jax.lax.bitcast_convert_type(b16, jnp.uint32) & jnp.uint32(0xFFFF0000)
        return au | bu

    def body(xa_ref, xb_ref, w_ref, b_ref, o_ref):
        w = w_ref[...]
        bias = b_ref[...]
        ta = jnp.maximum(
            jnp.dot(xa_ref[...], w, preferred_element_type=jnp.float32) + bias,
            0.0)
        tb = jnp.maximum(
            jnp.dot(xb_ref[...], w, preferred_element_type=jnp.float32) + bias,
            0.0)
        packed = jnp.concatenate([pack_half(ta, d), pack_half(tb, d)], axis=1)
        o_ref[...] = jax.lax.bitcast_convert_type(packed, jnp.int32)

    nh = n // 2
    grid = nh // rb
    return pl.pallas_call(
        body,
        grid=(grid,),
        in_specs=[
            pl.BlockSpec((rb, d), lambda i: (i, 0)),
            pl.BlockSpec((rb, d), lambda i, g=grid: (i + g, 0)),
            pl.BlockSpec((d, d), lambda i: (0, 0)),
            pl.BlockSpec((1, d), lambda i: (0, 0)),
        ],
        out_specs=pl.BlockSpec((rb, d), lambda i: (i, 0)),
        out_shape=jax.ShapeDtypeStruct((nh, d), jnp.int32),
    )(features, features, w, b.reshape(1, d))


def _out_matmul(nodef, agg, w2, b_rows):
    d = nodef.shape[1]
    u = w2.shape[1]
    rb = 1000
    assert b_rows % rb == 0

    def body(nf_ref, ag_ref, w_ref, o_ref):
        top = jnp.dot(nf_ref[...], w_ref[0:d, :], preferred_element_type=jnp.float32)
        bot = jnp.dot(ag_ref[...], w_ref[d:2 * d, :], preferred_element_type=jnp.float32)
        o_ref[...] = jnp.maximum(top + bot, 0.0)

    return pl.pallas_call(
        body,
        grid=(b_rows // rb,),
        in_specs=[
            pl.BlockSpec((rb, d), lambda i: (i, 0)),
            pl.BlockSpec((rb, d), lambda i: (i, 0)),
            pl.BlockSpec((2 * d, u), lambda i: (0, 0)),
        ],
        out_specs=pl.BlockSpec((rb, u), lambda i: (i, 0)),
        out_shape=jax.ShapeDtypeStruct((b_rows, u), jnp.float32),
    )(nodef, agg, w2)


def _sc_node_gather(node_idx, features, b_pad, d):
    bw = b_pad // _NW
    rows_per_gather = 64
    ng = bw // rows_per_gather
    assert bw % rows_per_gather == 0

    mesh = plsc.VectorSubcoreMesh(core_axis_name="c", subcore_axis_name="s")

    @functools.partial(
        pl.kernel,
        out_type=jax.ShapeDtypeStruct((b_pad, d), jnp.float32),
        mesh=mesh,
        compiler_params=pltpu.CompilerParams(use_tc_tiling_on_sc=False),
        scratch_types=[
            pltpu.VMEM((bw,), jnp.int32),
            pltpu.VMEM((bw, d), jnp.float32),
            pltpu.SemaphoreType.DMA,
        ],
    )
    def sc_kernel(node_hbm, feat_hbm, nodef_out, idx_nd, rows, sem):
        wid = lax.axis_index("s") * _NC + lax.axis_index("c")
        base = wid * bw
        pltpu.sync_copy(node_hbm.at[pl.ds(base, bw)], idx_nd)
        cps = []
        for j in range(ng):
            sl = pl.ds(j * rows_per_gather, rows_per_gather)
            cps.append(pltpu.make_async_copy(
                feat_hbm.at[idx_nd.at[sl]], rows.at[sl], sem))
        for cp in cps:
            cp.start()
        for cp in cps:
            cp.wait()
        pltpu.sync_copy(rows, nodef_out.at[pl.ds(base, bw)])

    return sc_kernel(node_idx, features)


def _sc_gather_mean(nb_idx, t_table, b_pad, d, k, nh):
    bw = b_pad // _NW
    c = 16
    nch = bw // c
    h = d // 2
    g = h // _LANES
    rows_per_gather = 128
    ng = (c * k) // rows_per_gather
    assert bw % c == 0 and (c * k) % rows_per_gather == 0 and nch % 2 == 0

    mesh = plsc.VectorSubcoreMesh(core_axis_name="c", subcore_axis_name="s")

    @functools.partial(
        pl.kernel,
        out_type=jax.ShapeDtypeStruct((b_pad, d), jnp.float32),
        mesh=mesh,
        compiler_params=pltpu.CompilerParams(use_tc_tiling_on_sc=False),
        scratch_types=[
            [pltpu.VMEM((c * k,), jnp.int32)] * 2,
            [pltpu.VMEM((c * k, h), jnp.int32)] * 2,
            [pltpu.VMEM((c, d), jnp.float32)] * 2,
            [pltpu.SemaphoreType.DMA] * 2,
            [pltpu.SemaphoreType.DMA] * 2,
        ],
    )
    def sc_kernel(nb_hbm, t_hbm, agg_out, idx_nb, rows, accbuf, sem, osem):
        wid = lax.axis_index("s") * _NC + lax.axis_index("c")
        base0 = wid * bw

        def gather_copies(p):
            cps = []
            for j in range(ng):
                sl = pl.ds(j * rows_per_gather, rows_per_gather)
                cps.append(pltpu.make_async_copy(
                    t_hbm.at[idx_nb[p].at[sl]], rows[p].at[sl], sem[p]))
            return cps

        def out_copies(p, base):
            return [
                pltpu.make_async_copy(accbuf[p],
                                      agg_out.at[pl.ds(base, c)], osem[p]),
            ]

        def stage(p, ch):
            base = base0 + ch * c
            pltpu.sync_copy(nb_hbm.at[pl.ds(base * k, c * k)], idx_nb[p])
            off = jnp.int32(1 - 2 * nh)
            for j in range(c * k // _LANES):
                sl = pl.ds(j * _LANES, _LANES)
                v = idx_nb[p][sl]
                idx_nb[p][sl] = 2 * v + jnp.where(v < nh, 0, off)

            @pl.when(ch >= 2)
            def _():
                for cp in out_copies(p, base):
                    cp.wait()

            for cp in gather_copies(p):
                cp.start()

        def drain(p):
            for cp in gather_copies(p):
                cp.wait()

        def compute(p, ch):
            base = base0 + ch * c

            def node_body(i, carry2):
                r0 = i * k
                himask = jnp.int32(-65536)

                def kacc(kk, accs):
                    lo = list(accs[:g])
                    hi = list(accs[g:])
                    for u in range(4):
                        r = r0 + kk * 4 + u
                        for gg in range(g):
                            w = rows[p][r, pl.ds(gg * _LANES, _LANES)]
                            lo[gg] = lo[gg] + lax.bitcast_convert_type(
                                w << 16, jnp.float32)
                            hi[gg] = hi[gg] + lax.bitcast_convert_type(
                                w & himask, jnp.float32)
                    return tuple(lo) + tuple(hi)

                zero = jnp.zeros((_LANES,), jnp.float32)
                accs = lax.fori_loop(0, k // 4, kacc, (zero,) * (2 * g))
                for gg in range(g):
                    accbuf[p][i, pl.ds(gg * _LANES, _LANES)] = (
                        accs[gg] * (1.0 / k))
                    accbuf[p][i, pl.ds(h + gg * _LANES, _LANES)] = (
                        accs[g + gg] * (1.0 / k))
                return carry2

            lax.fori_loop(0, c, node_body, 0)
            for cp in out_copies(p, base):
                cp.start()

        stage(0, 0)

        def loop_body(ch2, carry):
            ch = ch2 * 2
            stage(1, jnp.minimum(ch + 1, nch - 1))
            drain(0)
            compute(0, ch)
            stage(0, jnp.minimum(ch + 2, nch - 1))
            drain(1)
            compute(1, ch + 1)
            return carry

        lax.fori_loop(0, nch // 2, loop_body, 0)
        drain(0)
        for cp in out_copies(1, base0):
            cp.wait()

    return sc_kernel(nb_idx, t_table)


def kernel(features, node, neighbours, raw_features, W_dense, b_dense, neigh_weights):
    n, d = features.shape
    b, k = neighbours.shape

    chunk_rows = _NW * 16
    b_pad = ((b + chunk_rows - 1) // chunk_rows) * chunk_rows
    pad = b_pad - b
    node_flat = node.reshape(-1).astype(jnp.int32)
    nb_flat = neighbours.astype(jnp.int32)
    if pad:
        pad_nd = jnp.arange(pad, dtype=jnp.int32) % n
        pad_nb = (jnp.arange(pad * k, dtype=jnp.int32) % n).reshape(pad, k)
        node_flat = jnp.concatenate([node_flat, pad_nd])
        nb_flat = jnp.concatenate([nb_flat, pad_nb], axis=0)
    nb_flat = nb_flat.reshape(-1)

    nodef = _sc_node_gather(node_flat, features, b_pad, d)
    t_table = _table_mlp(features, W_dense, b_dense)
    t64 = t_table.reshape(n, d // 2)
    agg = _sc_gather_mean(nb_flat, t64, b_pad, d, k, n // 2)
    out = _out_matmul(nodef, agg, neigh_weights, b)
    return (out, raw_features)


# speedup vs baseline: 1.0100x; 1.0100x over previous
import functools

import jax
import jax.numpy as jnp
from jax import lax
from jax.experimental import pallas as pl
from jax.experimental.pallas import tpu as pltpu
from jax.experimental.pallas import tpu_sc as plsc

_NC = 2
_NS = 16
_NW = _NC * _NS
_LANES = 16


def _table_mlp(features, w, b):
    n, d = features.shape
    rb = 5000
    assert n % 2 == 0 and (n // 2) % rb == 0

    def pack_half(t, d):
        h = d // 2
        a16 = t[:, :h].astype(jnp.bfloat16).astype(jnp.float32)
        b16 = t[:, h:].astype(jnp.bfloat16).astype(jnp.float32)
        au = jax.lax.bitcast_convert_type(a16, jnp.uint32) >> 16
        bu = jax.lax.bitcast_convert_type(b16, jnp.uint32) & jnp.uint32(0xFFFF0000)
        return au | bu

    def body(xa_ref, xb_ref, w_ref, b_ref, o_ref):
        w = w_ref[...]
        bias = b_ref[...]
        ta = jnp.maximum(
            jnp.dot(xa_ref[...], w, preferred_element_type=jnp.float32) + bias,
            0.0)
        tb = jnp.maximum(
            jnp.dot(xb_ref[...], w, preferred_element_type=jnp.float32) + bias,
            0.0)
        packed = jnp.concatenate([pack_half(ta, d), pack_half(tb, d)], axis=1)
        o_ref[...] = jax.lax.bitcast_convert_type(packed, jnp.int32)

    nh = n // 2
    grid = nh // rb
    return pl.pallas_call(
        body,
        grid=(grid,),
        in_specs=[
            pl.BlockSpec((rb, d), lambda i: (i, 0)),
            pl.BlockSpec((rb, d), lambda i, g=grid: (i + g, 0)),
            pl.BlockSpec((d, d), lambda i: (0, 0)),
            pl.BlockSpec((1, d), lambda i: (0, 0)),
        ],
        out_specs=pl.BlockSpec((rb, d), lambda i: (i, 0)),
        out_shape=jax.ShapeDtypeStruct((nh, d), jnp.int32),
    )(features, features, w, b.reshape(1, d))


def _out_matmul(nodef, agg, w2, b_rows):
    d = nodef.shape[1]
    u = w2.shape[1]
    rb = 1000
    assert b_rows % rb == 0

    def body(nf_ref, ag_ref, w_ref, o_ref):
        top = jnp.dot(nf_ref[...], w_ref[0:d, :], preferred_element_type=jnp.float32)
        bot = jnp.dot(ag_ref[...], w_ref[d:2 * d, :], preferred_element_type=jnp.float32)
        o_ref[...] = jnp.maximum(top + bot, 0.0)

    return pl.pallas_call(
        body,
        grid=(b_rows // rb,),
        in_specs=[
            pl.BlockSpec((rb, d), lambda i: (i, 0)),
            pl.BlockSpec((rb, d), lambda i: (i, 0)),
            pl.BlockSpec((2 * d, u), lambda i: (0, 0)),
        ],
        out_specs=pl.BlockSpec((rb, u), lambda i: (i, 0)),
        out_shape=jax.ShapeDtypeStruct((b_rows, u), jnp.float32),
    )(nodef, agg, w2)


def _sc_node_gather(node_idx, features, b_pad, d):
    bw = b_pad // _NW
    rows_per_gather = 64
    ng = bw // rows_per_gather
    assert bw % rows_per_gather == 0

    mesh = plsc.VectorSubcoreMesh(core_axis_name="c", subcore_axis_name="s")

    @functools.partial(
        pl.kernel,
        out_type=jax.ShapeDtypeStruct((b_pad, d), jnp.float32),
        mesh=mesh,
        compiler_params=pltpu.CompilerParams(use_tc_tiling_on_sc=False),
        scratch_types=[
            pltpu.VMEM((bw,), jnp.int32),
            pltpu.VMEM((bw, d), jnp.float32),
            pltpu.SemaphoreType.DMA,
        ],
    )
    def sc_kernel(node_hbm, feat_hbm, nodef_out, idx_nd, rows, sem):
        wid = lax.axis_index("s") * _NC + lax.axis_index("c")
        base = wid * bw
        pltpu.sync_copy(node_hbm.at[pl.ds(base, bw)], idx_nd)
        cps = []
        for j in range(ng):
            sl = pl.ds(j * rows_per_gather, rows_per_gather)
            cps.append(pltpu.make_async_copy(
                feat_hbm.at[idx_nd.at[sl]], rows.at[sl], sem))
        for cp in cps:
            cp.start()
        for cp in cps:
            cp.wait()
        pltpu.sync_copy(rows, nodef_out.at[pl.ds(base, bw)])

    return sc_kernel(node_idx, features)


def _sc_gather_mean(nb_idx, t_table, b_pad, d, k):
    bw = b_pad // _NW
    c = 20
    nch = bw // c
    h = d // 2
    g = h // _LANES
    rows_per_gather = 128
    ng = (c * k) // rows_per_gather
    assert bw % c == 0 and (c * k) % rows_per_gather == 0 and nch % 2 == 0

    mesh = plsc.VectorSubcoreMesh(core_axis_name="c", subcore_axis_name="s")

    @functools.partial(
        pl.kernel,
        out_type=jax.ShapeDtypeStruct((b_pad, d), jnp.float32),
        mesh=mesh,
        compiler_params=pltpu.CompilerParams(use_tc_tiling_on_sc=False),
        scratch_types=[
            [pltpu.VMEM((c * k,), jnp.int32)] * 2,
            [pltpu.VMEM((c * k, h), jnp.int32)] * 2,
            [pltpu.VMEM((c, d), jnp.float32)] * 2,
            [pltpu.SemaphoreType.DMA] * 2,
            [pltpu.SemaphoreType.DMA] * 2,
        ],
    )
    def sc_kernel(nb_hbm, t_hbm, agg_out, idx_nb, rows, accbuf, sem, osem):
        wid = lax.axis_index("s") * _NC + lax.axis_index("c")
        base0 = wid * bw

        def gather_copies(p):
            cps = []
            for j in range(ng):
                sl = pl.ds(j * rows_per_gather, rows_per_gather)
                cps.append(pltpu.make_async_copy(
                    t_hbm.at[idx_nb[p].at[sl]], rows[p].at[sl], sem[p]))
            return cps

        def out_copies(p, base):
            return [
                pltpu.make_async_copy(accbuf[p],
                                      agg_out.at[pl.ds(base, c)], osem[p]),
            ]

        def stage(p, ch):
            base = base0 + ch * c
            pltpu.sync_copy(nb_hbm.at[pl.ds(base * k, c * k)], idx_nb[p])

            @pl.when(ch >= 2)
            def _():
                for cp in out_copies(p, base):
                    cp.wait()

            for cp in gather_copies(p):
                cp.start()

        def drain(p):
            for cp in gather_copies(p):
                cp.wait()

        def compute(p, ch):
            base = base0 + ch * c

            def node_body(i, carry2):
                r0 = i * k
                himask = jnp.int32(-65536)

                def kacc(kk, accs):
                    lo = list(accs[:g])
                    hi = list(accs[g:])
                    for u in range(4):
                        r = r0 + kk * 4 + u
                        for gg in range(g):
                            w = rows[p][r, pl.ds(gg * _LANES, _LANES)]
                            lo[gg] = lo[gg] + lax.bitcast_convert_type(
                                w << 16, jnp.float32)
                            hi[gg] = hi[gg] + lax.bitcast_convert_type(
                                w & himask, jnp.float32)
                    return tuple(lo) + tuple(hi)

                zero = jnp.zeros((_LANES,), jnp.float32)
                accs = lax.fori_loop(0, k // 4, kacc, (zero,) * (2 * g))
                for gg in range(g):
                    accbuf[p][i, pl.ds(gg * _LANES, _LANES)] = (
                        accs[gg] * (1.0 / k))
                    accbuf[p][i, pl.ds(h + gg * _LANES, _LANES)] = (
                        accs[g + gg] * (1.0 / k))
                return carry2

            lax.fori_loop(0, c, node_body, 0)
            for cp in out_copies(p, base):
                cp.start()

        stage(0, 0)

        def loop_body(ch2, carry):
            ch = ch2 * 2
            stage(1, jnp.minimum(ch + 1, nch - 1))
            drain(0)
            compute(0, ch)
            stage(0, jnp.minimum(ch + 2, nch - 1))
            drain(1)
            compute(1, ch + 1)
            return carry

        lax.fori_loop(0, nch // 2, loop_body, 0)
        drain(0)
        for cp in out_copies(1, base0):
            cp.wait()

    return sc_kernel(nb_idx, t_table)


def kernel(features, node, neighbours, raw_features, W_dense, b_dense, neigh_weights):
    n, d = features.shape
    b, k = neighbours.shape

    chunk_rows = _NW * 20 * 2
    b_pad = ((b + chunk_rows - 1) // chunk_rows) * chunk_rows
    pad = b_pad - b
    node_flat = node.reshape(-1).astype(jnp.int32)
    nb_flat = neighbours.astype(jnp.int32)
    if pad:
        pad_nd = jnp.arange(pad, dtype=jnp.int32) % n
        pad_nb = (jnp.arange(pad * k, dtype=jnp.int32) % n).reshape(pad, k)
        node_flat = jnp.concatenate([node_flat, pad_nd])
        nb_flat = jnp.concatenate([nb_flat, pad_nb], axis=0)
    nb_flat = nb_flat.reshape(-1)

    nodef = _sc_node_gather(node_flat, features, b_pad, d)
    t_table = _table_mlp(features, W_dense, b_dense)
    t64 = t_table.reshape(n, d // 2)
    nh = n // 2
    nb_flat = jnp.where(nb_flat < nh, 2 * nb_flat, 2 * (nb_flat - nh) + 1)
    agg = _sc_gather_mean(nb_flat, t64, b_pad, d, k)
    out = _out_matmul(nodef, agg, neigh_weights, b)
    return (out, raw_features)


# speedup vs baseline: 1.0145x; 1.0044x over previous
import functools

import jax
import jax.numpy as jnp
from jax import lax
from jax.experimental import pallas as pl
from jax.experimental.pallas import tpu as pltpu
from jax.experimental.pallas import tpu_sc as plsc

_NC = 2
_NS = 16
_NW = _NC * _NS
_LANES = 16


def _table_mlp(features, w, b):
    n, d = features.shape
    rb = 5000
    assert n % 2 == 0 and (n // 2) % rb == 0

    def pack_half(t, d):
        h = d // 2
        a16 = t[:, :h].astype(jnp.bfloat16).astype(jnp.float32)
        b16 = t[:, h:].astype(jnp.bfloat16).astype(jnp.float32)
        au = jax.lax.bitcast_convert_type(a16, jnp.uint32) >> 16
        bu = jax.lax.bitcast_convert_type(b16, jnp.uint32) & jnp.uint32(0xFFFF0000)
        return au | bu

    def body(xa_ref, xb_ref, w_ref, b_ref, o_ref):
        w = w_ref[...]
        bias = b_ref[...]
        ta = jnp.maximum(
            jnp.dot(xa_ref[...], w, preferred_element_type=jnp.float32) + bias,
            0.0)
        tb = jnp.maximum(
            jnp.dot(xb_ref[...], w, preferred_element_type=jnp.float32) + bias,
            0.0)
        packed = jnp.concatenate([pack_half(ta, d), pack_half(tb, d)], axis=1)
        o_ref[...] = jax.lax.bitcast_convert_type(packed, jnp.int32)

    nh = n // 2
    grid = nh // rb
    return pl.pallas_call(
        body,
        grid=(grid,),
        in_specs=[
            pl.BlockSpec((rb, d), lambda i: (i, 0)),
            pl.BlockSpec((rb, d), lambda i, g=grid: (i + g, 0)),
            pl.BlockSpec((d, d), lambda i: (0, 0)),
            pl.BlockSpec((1, d), lambda i: (0, 0)),
        ],
        out_specs=pl.BlockSpec((rb, d), lambda i: (i, 0)),
        out_shape=jax.ShapeDtypeStruct((nh, d), jnp.int32),
    )(features, features, w, b.reshape(1, d))


def _out_matmul(nodef, agg, w2, b_rows):
    d = nodef.shape[1]
    u = w2.shape[1]
    rb = 1000
    assert b_rows % rb == 0

    def body(nf_ref, ag_ref, w_ref, o_ref):
        top = jnp.dot(nf_ref[...], w_ref[0:d, :], preferred_element_type=jnp.float32)
        bot = jnp.dot(ag_ref[...], w_ref[d:2 * d, :], preferred_element_type=jnp.float32)
        o_ref[...] = jnp.maximum(top + bot, 0.0)

    return pl.pallas_call(
        body,
        grid=(b_rows // rb,),
        in_specs=[
            pl.BlockSpec((rb, d), lambda i: (i, 0)),
            pl.BlockSpec((rb, d), lambda i: (i, 0)),
            pl.BlockSpec((2 * d, u), lambda i: (0, 0)),
        ],
        out_specs=pl.BlockSpec((rb, u), lambda i: (i, 0)),
        out_shape=jax.ShapeDtypeStruct((b_rows, u), jnp.float32),
    )(nodef, agg, w2)


def _sc_node_gather(node_idx, features, b, d):
    bw = ((b + _NW - 1) // _NW + 63) // 64 * 64
    rows_per_gather = 64
    ng = bw // rows_per_gather
    assert b >= bw and (b - bw) % 8 == 0

    mesh = plsc.VectorSubcoreMesh(core_axis_name="c", subcore_axis_name="s")

    @functools.partial(
        pl.kernel,
        out_type=jax.ShapeDtypeStruct((b, d), jnp.float32),
        mesh=mesh,
        compiler_params=pltpu.CompilerParams(use_tc_tiling_on_sc=False),
        scratch_types=[
            pltpu.VMEM((bw,), jnp.int32),
            pltpu.VMEM((bw, d), jnp.float32),
            pltpu.SemaphoreType.DMA,
        ],
    )
    def sc_kernel(node_hbm, feat_hbm, nodef_out, idx_nd, rows, sem):
        wid = lax.axis_index("s") * _NC + lax.axis_index("c")
        base = jnp.minimum(wid * bw, b - bw)
        pltpu.sync_copy(node_hbm.at[pl.ds(base, bw)], idx_nd)
        cps = []
        for j in range(ng):
            sl = pl.ds(j * rows_per_gather, rows_per_gather)
            cps.append(pltpu.make_async_copy(
                feat_hbm.at[idx_nd.at[sl]], rows.at[sl], sem))
        for cp in cps:
            cp.start()
        for cp in cps:
            cp.wait()
        pltpu.sync_copy(rows, nodef_out.at[pl.ds(base, bw)])

    return sc_kernel(node_idx, features)


def _sc_gather_mean(nb_idx, t_table, b, d, k):
    c = 20
    bw = ((b + _NW - 1) // _NW + 2 * c - 1) // (2 * c) * (2 * c)
    nch = bw // c
    assert b >= bw and ((b - c) * k) % 8 == 0 and (c * k) % 8 == 0
    h = d // 2
    g = h // _LANES
    rows_per_gather = 128
    ng = (c * k) // rows_per_gather
    assert bw % c == 0 and (c * k) % rows_per_gather == 0 and nch % 2 == 0

    mesh = plsc.VectorSubcoreMesh(core_axis_name="c", subcore_axis_name="s")

    @functools.partial(
        pl.kernel,
        out_type=jax.ShapeDtypeStruct((b, d), jnp.float32),
        mesh=mesh,
        compiler_params=pltpu.CompilerParams(use_tc_tiling_on_sc=False),
        scratch_types=[
            [pltpu.VMEM((c * k,), jnp.int32)] * 2,
            [pltpu.VMEM((c * k, h), jnp.int32)] * 2,
            [pltpu.VMEM((c, d), jnp.float32)] * 2,
            [pltpu.SemaphoreType.DMA] * 2,
            [pltpu.SemaphoreType.DMA] * 2,
        ],
    )
    def sc_kernel(nb_hbm, t_hbm, agg_out, idx_nb, rows, accbuf, sem, osem):
        wid = lax.axis_index("s") * _NC + lax.axis_index("c")
        base0 = wid * bw

        def gather_copies(p):
            cps = []
            for j in range(ng):
                sl = pl.ds(j * rows_per_gather, rows_per_gather)
                cps.append(pltpu.make_async_copy(
                    t_hbm.at[idx_nb[p].at[sl]], rows[p].at[sl], sem[p]))
            return cps

        def out_copies(p, base):
            return [
                pltpu.make_async_copy(accbuf[p],
                                      agg_out.at[pl.ds(base, c)], osem[p]),
            ]

        def stage(p, ch):
            base = jnp.minimum(base0 + ch * c, b - c)
            pltpu.sync_copy(nb_hbm.at[pl.ds(base * k, c * k)], idx_nb[p])

            @pl.when(ch >= 2)
            def _():
                for cp in out_copies(p, base):
                    cp.wait()

            for cp in gather_copies(p):
                cp.start()

        def drain(p):
            for cp in gather_copies(p):
                cp.wait()

        def compute(p, ch):
            base = jnp.minimum(base0 + ch * c, b - c)

            def node_body(i, carry2):
                r0 = i * k
                himask = jnp.int32(-65536)

                def kacc(kk, accs):
                    lo = list(accs[:g])
                    hi = list(accs[g:])
                    for u in range(4):
                        r = r0 + kk * 4 + u
                        for gg in range(g):
                            w = rows[p][r, pl.ds(gg * _LANES, _LANES)]
                            lo[gg] = lo[gg] + lax.bitcast_convert_type(
                                w << 16, jnp.float32)
                            hi[gg] = hi[gg] + lax.bitcast_convert_type(
                                w & himask, jnp.float32)
                    return tuple(lo) + tuple(hi)

                zero = jnp.zeros((_LANES,), jnp.float32)
                accs = lax.fori_loop(0, k // 4, kacc, (zero,) * (2 * g))
                for gg in range(g):
                    accbuf[p][i, pl.ds(gg * _LANES, _LANES)] = (
                        accs[gg] * (1.0 / k))
                    accbuf[p][i, pl.ds(h + gg * _LANES, _LANES)] = (
                        accs[g + gg] * (1.0 / k))
                return carry2

            lax.fori_loop(0, c, node_body, 0)
            for cp in out_copies(p, base):
                cp.start()

        stage(0, 0)

        def loop_body(ch2, carry):
            ch = ch2 * 2
            stage(1, ch + 1)
            drain(0)
            compute(0, ch)
            stage(0, ch + 2)
            drain(1)
            compute(1, ch + 1)
            return carry

        lax.fori_loop(0, nch // 2, loop_body, 0)
        drain(0)
        for cp in out_copies(1, base0):
            cp.wait()

    return sc_kernel(nb_idx, t_table)


def kernel(features, node, neighbours, raw_features, W_dense, b_dense, neigh_weights):
    n, d = features.shape
    b, k = neighbours.shape

    node_flat = node.reshape(-1).astype(jnp.int32)
    nb_flat = neighbours.astype(jnp.int32).reshape(-1)

    nodef = _sc_node_gather(node_flat, features, b, d)
    t_table = _table_mlp(features, W_dense, b_dense)
    t64 = t_table.reshape(n, d // 2)
    nh = n // 2
    nb_flat = jnp.where(nb_flat < nh, 2 * nb_flat, 2 * (nb_flat - nh) + 1)
    agg = _sc_gather_mean(nb_flat, t64, b, d, k)
    out = _out_matmul(nodef, agg, neigh_weights, b)
    return (out, raw_features)
